# baseline (speedup 1.0000x reference)
import functools

import jax
import jax.numpy as jnp
from jax import lax
from jax.experimental import pallas as pl
from jax.experimental.pallas import tpu as pltpu
from jax.experimental.pallas import tpu_sc as plsc

_F = 26
_V = 100000
_D = 32
_B = 4096

_INFO = plsc.get_sparse_core_info()
_NC = _INFO.num_cores
_NS = _INFO.num_subcores
_L = _INFO.num_lanes
_NW = _NC * _NS

_TOTAL = _B * _F
_PER_W = _TOTAL // _NW
_CHUNK = 128
_NCHUNK = _PER_W // _CHUNK
_VPC = _CHUNK // _L


def _gather_body(tab_hbm, idx_hbm, out_hbm, raw_v, idx2_v, rows_v, sem):
    wid = lax.axis_index("s") * _NC + lax.axis_index("c")
    base = wid * _PER_W

    pltpu.sync_copy(idx_hbm.at[pl.ds(base, _PER_W)], raw_v)

    lane = lax.iota(jnp.int32, 16)

    def issue_chunk(c, carry):
        def flatten_vec(l, inner):
            off = c * _CHUNK + l * _L
            pos = base + off + lane
            f = lax.rem(pos, _F)
            idx2_v[c, pl.ds(l * _L, _L)] = raw_v[pl.ds(off, _L)] + f * _V
            return inner

        lax.fori_loop(0, _VPC, flatten_vec, 0)
        pltpu.make_async_copy(
            tab_hbm.at[idx2_v.at[c]],
            rows_v.at[pl.ds(c * _CHUNK, _CHUNK)],
            sem,
        ).start()
        return carry

    lax.fori_loop(0, _NCHUNK, issue_chunk, 0)

    def drain_chunk(c, carry):
        pltpu.make_async_copy(
            tab_hbm.at[idx2_v.at[c]],
            rows_v.at[pl.ds(c * _CHUNK, _CHUNK)],
            sem,
        ).wait()
        return carry

    lax.fori_loop(0, _NCHUNK, drain_chunk, 0)

    pltpu.sync_copy(rows_v, out_hbm.at[pl.ds(base, _PER_W)])


_sc_gather = functools.partial(
    pl.kernel,
    mesh=plsc.VectorSubcoreMesh(core_axis_name="c", subcore_axis_name="s"),
    compiler_params=pltpu.CompilerParams(use_tc_tiling_on_sc=False),
    out_type=jax.ShapeDtypeStruct((_TOTAL, _D), jnp.float32),
    scratch_types=[
        pltpu.VMEM((_PER_W,), jnp.int32),
        pltpu.VMEM((_NCHUNK, _CHUNK), jnp.int32),
        pltpu.VMEM((_PER_W, _D), jnp.float32),
        pltpu.SemaphoreType.DMA,
    ],
)(_gather_body)


@jax.jit
def kernel(indices, tables):
    idx_flat = indices.reshape(_TOTAL)
    tab_flat = tables.reshape(_F * _V, _D)
    out = _sc_gather(tab_flat, idx_flat)
    return out.reshape(_B, _F, _D)
